# baseline (speedup 1.0000x reference)
import jax
import jax.numpy as jnp
from jax import lax
from jax.experimental import pallas as pl
from jax.experimental.pallas import tpu as pltpu
from jax.experimental.pallas import tpu_sc as plsc

N_USERS = 5000
N_ITEMS = 5000
N = N_USERS + N_ITEMS
EMB = 128
HALF = EMB // 2
LAYERS = 3
E = 320000

NC = 2
NS = 16
EPT = E // NS
CH = 400
NCHUNK = EPT // CH
RPT = N // NS


def _body(ed_hbm, x0_hbm, y1, y2, y3,
          acc, eb0, eb1, gb0, gb1, ie0, ie1, ge0, ge1):
    c = lax.axis_index("c")
    s = lax.axis_index("s")
    r0 = s * RPT
    ebufs, gbufs = (eb0, eb1), (gb0, gb1)
    isems, gsems = (ie0, ie1), (ge0, ge1)

    zeros16 = jnp.zeros((16,), jnp.float32)

    def zero_gbuf(i, carry):
        for cb in range(HALF // 16):
            gb0[i, pl.ds(cb * 16, 16)] = zeros16
        return carry

    def layer(xin, yout):
        lax.fori_loop(0, CH, zero_gbuf, 0)
        done = 0
        while done < RPT:
            step = min(CH, RPT - done)
            pltpu.sync_copy(gb0.at[pl.ds(0, step)],
                            acc.at[pl.ds(r0 + done, step)])
            done += step
        plsc.subcore_barrier()

        def issue_idx(kk, b):
            pltpu.async_copy(ed_hbm.at[s * NCHUNK + kk], ebufs[b], isems[b])

        def wait_idx(b):
            pltpu.make_async_copy(ed_hbm.at[0], ebufs[b], isems[b]).wait()

        def issue_gather(b):
            pltpu.async_copy(xin.at[ebufs[b].at[0]], gbufs[b], gsems[b])

        def wait_gather(b):
            pltpu.make_async_copy(xin.at[ebufs[b].at[0]], gbufs[b],
                                  gsems[b]).wait()

        issue_idx(0, 0)
        issue_idx(1, 1)
        wait_idx(0)
        issue_gather(0)

        def pair(k, carry):
            for b in range(2):
                kk = k + b
                wait_gather(b)

                @pl.when(kk + 1 < NCHUNK)
                def _():
                    wait_idx(1 - b)
                    issue_gather(1 - b)

                gbuf = gbufs[b]
                wbits = ebufs[b]

                def scale(g, inner):
                    wvec = lax.bitcast_convert_type(
                        wbits[2, pl.ds(g * 16, 16)], jnp.float32)
                    for j in range(16):
                        e = g * 16 + j
                        wj = wvec.at[jnp.full((16,), j, jnp.int32)].get(
                            mode="promise_in_bounds")
                        for cb in range(HALF // 16):
                            sl = pl.ds(cb * 16, 16)
                            gbuf[e, sl] = gbuf[e, sl] * wj
                    return inner

                lax.fori_loop(0, CH // 16, scale, 0, unroll=4)
                pltpu.sync_copy(gbuf, acc.at[ebufs[b].at[1]], add=True)

                @pl.when(kk + 2 < NCHUNK)
                def _():
                    issue_idx(kk + 2, b)
            return carry

        lax.fori_loop(0, NCHUNK // 2, lambda i, cy: pair(i * 2, cy), 0)
        plsc.subcore_barrier()
        pltpu.sync_copy(acc.at[pl.ds(r0, RPT)], yout.at[c, pl.ds(r0, RPT)])
        plsc.subcore_barrier()

    layer(x0_hbm.at[c], y1)
    layer(y1.at[c], y2)
    layer(y2.at[c], y3)


@jax.jit
def _propagate(edata, x0_halves):
    out3 = [jax.ShapeDtypeStruct((NC, N, HALF), jnp.float32)] * LAYERS
    run = pl.kernel(
        _body,
        out_type=out3,
        mesh=plsc.VectorSubcoreMesh(core_axis_name="c", subcore_axis_name="s"),
        scratch_types=[
            pltpu.VMEM_SHARED((N, HALF), jnp.float32),
            pltpu.VMEM((3, CH), jnp.int32),
            pltpu.VMEM((3, CH), jnp.int32),
            pltpu.VMEM((CH, HALF), jnp.float32),
            pltpu.VMEM((CH, HALF), jnp.float32),
            pltpu.SemaphoreType.DMA,
            pltpu.SemaphoreType.DMA,
            pltpu.SemaphoreType.DMA,
            pltpu.SemaphoreType.DMA,
        ],
        compiler_params=pltpu.CompilerParams(use_tc_tiling_on_sc=False),
    )
    return run(edata, x0_halves)


def kernel(edge_index, edge_weight, user_emb, item_emb):
    x0 = jnp.concatenate([user_emb, item_emb], axis=0)
    x0_halves = jnp.stack([x0[:, :HALF], x0[:, HALF:]], axis=0)
    src = edge_index[0].astype(jnp.int32)
    dst = edge_index[1].astype(jnp.int32)
    wbits = lax.bitcast_convert_type(edge_weight.astype(jnp.float32),
                                     jnp.int32)
    edata = (jnp.stack([src, dst, wbits])
             .reshape(3, NS, NCHUNK, CH)
             .transpose(1, 2, 0, 3)
             .reshape(NS * NCHUNK, 3, CH))
    ys = _propagate(edata, x0_halves)
    outs = tuple(y.transpose(1, 0, 2).reshape(N, EMB) for y in ys)
    return (x0,) + outs


# speedup vs baseline: 1.1483x; 1.1483x over previous
import jax
import jax.numpy as jnp
from jax import lax
from jax.experimental import pallas as pl
from jax.experimental.pallas import tpu as pltpu
from jax.experimental.pallas import tpu_sc as plsc

N_USERS = 5000
N_ITEMS = 5000
N = N_USERS + N_ITEMS
EMB = 128
HALF = EMB // 2
LAYERS = 3
E = 320000

NC = 2
NS = 16
EPT = E // NS
CH = 400
NCHUNK = EPT // CH
RPT = N // NS


def _body(ed_hbm, x0_hbm, y1, y2, y3,
          acc, eb0, eb1, gb0, gb1, db0, db1,
          ie0, ie1, ge0, ge1, se0, se1):
    c = lax.axis_index("c")
    s = lax.axis_index("s")
    r0 = s * RPT
    ebufs, gbufs, dbufs = (eb0, eb1), (gb0, gb1), (db0, db1)
    isems, gsems, ssems = (ie0, ie1), (ge0, ge1), (se0, se1)

    zeros16 = jnp.zeros((16,), jnp.float32)

    def zero_gbuf(i, carry):
        for cb in range(HALF // 16):
            gb0[i, pl.ds(cb * 16, 16)] = zeros16
        return carry

    def layer(xin, yout):
        lax.fori_loop(0, CH, zero_gbuf, 0)
        done = 0
        while done < RPT:
            step = min(CH, RPT - done)
            pltpu.sync_copy(gb0.at[pl.ds(0, step)],
                            acc.at[pl.ds(r0 + done, step)])
            done += step
        plsc.subcore_barrier()

        def issue_idx(kk, b):
            pltpu.async_copy(ed_hbm.at[s * NCHUNK + kk], ebufs[b], isems[b])

        def wait_idx(b):
            pltpu.make_async_copy(ed_hbm.at[0], ebufs[b], isems[b]).wait()

        def issue_gather(b):
            pltpu.async_copy(xin.at[ebufs[b].at[0]], gbufs[b], gsems[b])

        def wait_gather(b):
            pltpu.make_async_copy(xin.at[ebufs[b].at[0]], gbufs[b],
                                  gsems[b]).wait()

        def issue_scatter(b):
            pltpu.async_copy(gbufs[b], acc.at[dbufs[b]], ssems[b], add=True)

        def wait_scatter(b):
            pltpu.make_async_copy(gbufs[b], acc.at[dbufs[b]],
                                  ssems[b]).wait()

        issue_idx(0, 0)
        issue_idx(1, 1)
        wait_idx(0)
        issue_gather(0)

        def pair(k, carry):
            for b in range(2):
                kk = k + b
                wait_gather(b)

                @pl.when(kk >= 1)
                def _():
                    wait_scatter(1 - b)

                @pl.when(kk + 1 < NCHUNK)
                def _():
                    wait_idx(1 - b)
                    issue_gather(1 - b)

                gbuf = gbufs[b]
                wbits = ebufs[b]

                def scale(g, inner):
                    wvec = lax.bitcast_convert_type(
                        wbits[2, pl.ds(g * 16, 16)], jnp.float32)
                    for j in range(16):
                        e = g * 16 + j
                        wj = wvec.at[jnp.full((16,), j, jnp.int32)].get(
                            mode="promise_in_bounds")
                        for cb in range(HALF // 16):
                            sl = pl.ds(cb * 16, 16)
                            gbuf[e, sl] = gbuf[e, sl] * wj
                    return inner

                for q in range(CH // 16):
                    dbufs[b][pl.ds(q * 16, 16)] = ebufs[b][1, pl.ds(q * 16, 16)]
                lax.fori_loop(0, CH // 16, scale, 0, unroll=2)
                issue_scatter(b)

                @pl.when(kk + 2 < NCHUNK)
                def _():
                    issue_idx(kk + 2, b)
            return carry

        lax.fori_loop(0, NCHUNK // 2, lambda i, cy: pair(i * 2, cy), 0)
        wait_scatter((NCHUNK - 1) % 2)
        plsc.subcore_barrier()
        pltpu.sync_copy(acc.at[pl.ds(r0, RPT)], yout.at[c, pl.ds(r0, RPT)])
        plsc.subcore_barrier()

    layer(x0_hbm.at[c], y1)
    layer(y1.at[c], y2)
    layer(y2.at[c], y3)


@jax.jit
def _propagate(edata, x0_halves):
    out3 = [jax.ShapeDtypeStruct((NC, N, HALF), jnp.float32)] * LAYERS
    run = pl.kernel(
        _body,
        out_type=out3,
        mesh=plsc.VectorSubcoreMesh(core_axis_name="c", subcore_axis_name="s"),
        scratch_types=[
            pltpu.VMEM_SHARED((N, HALF), jnp.float32),
            pltpu.VMEM((3, CH), jnp.int32),
            pltpu.VMEM((3, CH), jnp.int32),
            pltpu.VMEM((CH, HALF), jnp.float32),
            pltpu.VMEM((CH, HALF), jnp.float32),
            pltpu.VMEM((CH,), jnp.int32),
            pltpu.VMEM((CH,), jnp.int32),
            pltpu.SemaphoreType.DMA,
            pltpu.SemaphoreType.DMA,
            pltpu.SemaphoreType.DMA,
            pltpu.SemaphoreType.DMA,
            pltpu.SemaphoreType.DMA,
            pltpu.SemaphoreType.DMA,
        ],
        compiler_params=pltpu.CompilerParams(use_tc_tiling_on_sc=False),
    )
    return run(edata, x0_halves)


def kernel(edge_index, edge_weight, user_emb, item_emb):
    x0 = jnp.concatenate([user_emb, item_emb], axis=0)
    x0_halves = jnp.stack([x0[:, :HALF], x0[:, HALF:]], axis=0)
    src = edge_index[0].astype(jnp.int32)
    dst = edge_index[1].astype(jnp.int32)
    wbits = lax.bitcast_convert_type(edge_weight.astype(jnp.float32),
                                     jnp.int32)
    edata = (jnp.stack([src, dst, wbits])
             .reshape(3, NS, NCHUNK, CH)
             .transpose(1, 2, 0, 3)
             .reshape(NS * NCHUNK, 3, CH))
    ys = _propagate(edata, x0_halves)
    outs = tuple(y.transpose(1, 0, 2).reshape(N, EMB) for y in ys)
    return (x0,) + outs


# speedup vs baseline: 1.3496x; 1.1752x over previous
import jax
import jax.numpy as jnp
from jax import lax
from jax.experimental import pallas as pl
from jax.experimental.pallas import tpu as pltpu
from jax.experimental.pallas import tpu_sc as plsc

N_USERS = 5000
N_ITEMS = 5000
N = N_USERS + N_ITEMS
EMB = 128
HALF = EMB // 2
LAYERS = 3
E = 320000

NC = 2
NS = 16
EPT = E // NS
CH = 400
NCHUNK = EPT // CH
RPT = N // NS
ZCH = 400


def _body(src_hbm, dst_hbm, w_hbm, user_hbm, item_hbm,
          x0f, y1f, y2f, y3f, x0h, h1, h2,
          acc, sb0, sb1, db0, db1, wb0, wb1, gb0, gb1, zbuf,
          is0, is1, ws0, ws1, ds0, ds1, ge0, ge1, se0, se1):
    c = lax.axis_index("c")
    s = lax.axis_index("s")
    r0 = s * RPT
    c0 = c * HALF
    sbufs, dbufs, wbufs = (sb0, sb1), (db0, db1), (wb0, wb1)
    gbufs = (gb0, gb1)
    isems, wsems, dsems = (is0, is1), (ws0, ws1), (ds0, ds1)
    gsems, ssems = (ge0, ge1), (se0, se1)

    zeros16 = jnp.zeros((16,), jnp.float32)

    def zero_zbuf(i, carry):
        for cb in range(HALF // 16):
            zbuf[i, pl.ds(cb * 16, 16)] = zeros16
        return carry

    lax.fori_loop(0, ZCH, zero_zbuf, 0)

    def zero_acc_rows():
        done = 0
        while done < RPT:
            step = min(ZCH, RPT - done)
            pltpu.sync_copy(zbuf.at[pl.ds(0, step)],
                            acc.at[pl.ds(r0 + done, step)])
            done += step

    def stage(emb, roff):
        done = 0
        while done < RPT:
            step = min(ZCH, RPT - done)
            pltpu.sync_copy(
                emb.at[pl.ds(roff + done, step), pl.ds(c0, HALF)],
                gb0.at[pl.ds(0, step)])
            pltpu.sync_copy(gb0.at[pl.ds(0, step)],
                            x0h.at[c, pl.ds(r0 + done, step)])
            pltpu.sync_copy(gb0.at[pl.ds(0, step)],
                            x0f.at[pl.ds(r0 + done, step), pl.ds(c0, HALF)])
            done += step

    @pl.when(s < NS // 2)
    def _():
        stage(user_hbm, r0)

    @pl.when(s >= NS // 2)
    def _():
        stage(item_hbm, r0 - N_USERS)

    zero_acc_rows()
    plsc.subcore_barrier()

    def layer(xin, yfull, hout):

        def issue_src(kk, b):
            off = s * EPT + kk * CH
            pltpu.async_copy(src_hbm.at[pl.ds(off, CH)], sbufs[b], isems[b])

        def issue_w(kk, b):
            off = s * EPT + kk * CH
            pltpu.async_copy(w_hbm.at[pl.ds(off, CH)], wbufs[b], wsems[b])

        def issue_dst(kk, b):
            off = s * EPT + kk * CH
            pltpu.async_copy(dst_hbm.at[pl.ds(off, CH)], dbufs[b], dsems[b])

        def wait_src(b):
            pltpu.make_async_copy(src_hbm.at[pl.ds(0, CH)], sbufs[b],
                                  isems[b]).wait()

        def wait_w(b):
            pltpu.make_async_copy(w_hbm.at[pl.ds(0, CH)], wbufs[b],
                                  wsems[b]).wait()

        def wait_dst(b):
            pltpu.make_async_copy(dst_hbm.at[pl.ds(0, CH)], dbufs[b],
                                  dsems[b]).wait()

        def issue_gather(b):
            pltpu.async_copy(xin.at[sbufs[b]], gbufs[b], gsems[b])

        def wait_gather(b):
            pltpu.make_async_copy(xin.at[sbufs[b]], gbufs[b],
                                  gsems[b]).wait()

        def issue_scatter(b):
            pltpu.async_copy(gbufs[b], acc.at[dbufs[b]], ssems[b], add=True)

        def wait_scatter(b):
            pltpu.make_async_copy(gbufs[b], acc.at[dbufs[b]],
                                  ssems[b]).wait()

        issue_src(0, 0)
        issue_w(0, 0)
        issue_src(1, 1)
        issue_w(1, 1)
        issue_dst(0, 0)
        wait_src(0)
        issue_gather(0)

        def pair(k, carry):
            for b in range(2):
                kk = k + b
                wait_gather(b)

                @pl.when(kk >= 1)
                def _():
                    wait_scatter(1 - b)

                @pl.when(kk + 1 < NCHUNK)
                def _():
                    issue_dst(kk + 1, 1 - b)
                    wait_src(1 - b)
                    issue_gather(1 - b)

                gbuf = gbufs[b]
                wbuf = wbufs[b]
                wait_w(b)

                def scale(g, inner):
                    wvec = wbuf[pl.ds(g * 16, 16)]
                    for j in range(16):
                        e = g * 16 + j
                        wj = wvec.at[jnp.full((16,), j, jnp.int32)].get(
                            mode="promise_in_bounds")
                        for cb in range(HALF // 16):
                            sl = pl.ds(cb * 16, 16)
                            gbuf[e, sl] = gbuf[e, sl] * wj
                    return inner

                lax.fori_loop(0, CH // 16, scale, 0, unroll=2)
                wait_dst(b)
                issue_scatter(b)

                @pl.when(kk + 2 < NCHUNK)
                def _():
                    issue_src(kk + 2, b)
                    issue_w(kk + 2, b)
            return carry

        lax.fori_loop(0, NCHUNK // 2, lambda i, cy: pair(i * 2, cy), 0)
        wait_scatter((NCHUNK - 1) % 2)
        plsc.subcore_barrier()

        done = 0
        while done < RPT:
            step = min(ZCH, RPT - done)
            rows = pl.ds(r0 + done, step)
            if hout is not None:
                pltpu.sync_copy(acc.at[rows], hout.at[c, rows])
            pltpu.sync_copy(acc.at[rows],
                            yfull.at[rows, pl.ds(c0, HALF)])
            done += step
        if hout is not None:
            zero_acc_rows()
        plsc.subcore_barrier()

    layer(x0h.at[c], y1f, h1)
    layer(h1.at[c], y2f, h2)
    layer(h2.at[c], y3f, None)


@jax.jit
def _propagate(src, dst, w, user_emb, item_emb):
    f32 = jnp.float32
    out_type = [
        jax.ShapeDtypeStruct((N, EMB), f32),
        jax.ShapeDtypeStruct((N, EMB), f32),
        jax.ShapeDtypeStruct((N, EMB), f32),
        jax.ShapeDtypeStruct((N, EMB), f32),
        jax.ShapeDtypeStruct((NC, N, HALF), f32),
        jax.ShapeDtypeStruct((NC, N, HALF), f32),
        jax.ShapeDtypeStruct((NC, N, HALF), f32),
    ]
    run = pl.kernel(
        _body,
        out_type=out_type,
        mesh=plsc.VectorSubcoreMesh(core_axis_name="c", subcore_axis_name="s"),
        scratch_types=[
            pltpu.VMEM_SHARED((N, HALF), f32),
            pltpu.VMEM((CH,), jnp.int32),
            pltpu.VMEM((CH,), jnp.int32),
            pltpu.VMEM((CH,), jnp.int32),
            pltpu.VMEM((CH,), jnp.int32),
            pltpu.VMEM((CH,), f32),
            pltpu.VMEM((CH,), f32),
            pltpu.VMEM((CH, HALF), f32),
            pltpu.VMEM((CH, HALF), f32),
            pltpu.VMEM((ZCH, HALF), f32),
            pltpu.SemaphoreType.DMA,
            pltpu.SemaphoreType.DMA,
            pltpu.SemaphoreType.DMA,
            pltpu.SemaphoreType.DMA,
            pltpu.SemaphoreType.DMA,
            pltpu.SemaphoreType.DMA,
            pltpu.SemaphoreType.DMA,
            pltpu.SemaphoreType.DMA,
            pltpu.SemaphoreType.DMA,
            pltpu.SemaphoreType.DMA,
        ],
        compiler_params=pltpu.CompilerParams(use_tc_tiling_on_sc=False),
    )
    return run(src, dst, w, user_emb, item_emb)


def kernel(edge_index, edge_weight, user_emb, item_emb):
    src = edge_index[0].astype(jnp.int32)
    dst = edge_index[1].astype(jnp.int32)
    w = edge_weight.astype(jnp.float32)
    outs = _propagate(src, dst, w, user_emb.astype(jnp.float32),
                      item_emb.astype(jnp.float32))
    return tuple(outs[:4])
